# baseline (speedup 1.0000x reference)
import functools

import jax
import jax.numpy as jnp
from jax import lax
from jax.experimental import pallas as pl
from jax.experimental.pallas import tpu as pltpu
from jax.experimental.pallas import tpu_sc as plsc

_PEOPLE = 30
_JOINTS = 17
_KP = _PEOPLE * _JOINTS
_KP_WORDS = 2 * _KP
_KP_ROW = 1024
_GAT = 544
_EPS = 1e-6


@functools.lru_cache(maxsize=None)
def _build(num_images, tags_per_image):
    mesh = plsc.VectorSubcoreMesh(core_axis_name="c", subcore_axis_name="s")
    per_core = num_images // 2

    def body(tags_ref, kp_ref, out_ref, kp_v, idx_v, gat_v,
             mean_v, val_v, row_v, sem):
        cid = lax.axis_index("c")
        sid = lax.axis_index("s")
        img = cid * per_core + sid
        lanes = lax.iota(jnp.int32, 16)

        @pl.when(sid < per_core)
        def _():
            pltpu.sync_copy(kp_ref.at[img], kp_v)

            off = jnp.full((16,), img * tags_per_image, jnp.int32)

            @pl.loop(0, 32)
            def _(c):
                flat = lanes + c * 16
                a_idx = jnp.minimum(flat * 2, _KP_WORDS - 2)
                kv = plsc.load_gather(kp_v, [a_idx])
                idx_v[pl.ds(c * 16, 16)] = kv + off

            copies = []
            for b in range(4):
                copies.append(pltpu.async_copy(
                    tags_ref.at[idx_v.at[pl.ds(b * 128, 128)]],
                    gat_v.at[pl.ds(b * 128, 128)], sem))
            for cp in copies:
                cp.wait()

            pull_acc = jnp.zeros((16,), jnp.float32)
            nval_acc = jnp.zeros((16,), jnp.float32)
            fz = jnp.zeros((16,), jnp.float32)
            for pv in range(2):
                p0 = lanes + pv * 16
                pmask = p0 < _PEOPLE
                base = p0 * _JOINTS

                @pl.loop(0, _JOINTS, init_carry=(fz, fz, fz))
                def sums(j, carry):
                    s1, s2, cnt = carry
                    g = plsc.load_gather(gat_v, [base + j])
                    a_vis = jnp.minimum(
                        (base + j) * 2 + 1, _KP_WORDS - 1)
                    vi = plsc.load_gather(kp_v, [a_vis])
                    vb = (vi > 0) & pmask
                    s1 = s1 + jnp.where(vb, g, 0.0)
                    s2 = s2 + jnp.where(vb, g * g, 0.0)
                    cnt = cnt + jnp.where(vb, 1.0, 0.0)
                    return s1, s2, cnt

                s1, s2, cnt = sums
                safe = jnp.maximum(cnt, 1.0)
                mean = s1 / safe
                valid = cnt > 0.0
                pull_acc = pull_acc + jnp.where(
                    valid, (s2 - mean * s1) / safe, 0.0)
                nval_acc = nval_acc + jnp.where(valid, 1.0, 0.0)
                mean_v[pl.ds(pv * 16, 16)] = mean
                val_v[pl.ds(pv * 16, 16)] = jnp.where(valid, 1.0, 0.0)

            @pl.loop(1, _PEOPLE, init_carry=fz)
            def push_acc(q, acc):
                qs = jnp.broadcast_to(q, (16,))
                mq = plsc.load_gather(mean_v, [qs])
                vq = plsc.load_gather(val_v, [qs])
                for pv in range(2):
                    p0 = lanes + pv * 16
                    mp = mean_v[pl.ds(pv * 16, 16)]
                    vp = val_v[pl.ds(pv * 16, 16)]
                    d2 = (mp - mq) * (mp - mq)
                    sel = ((p0 < q) & (d2 != 0.0)
                           & (vp > 0.0) & (vq > 0.0))
                    acc = acc + jnp.where(sel, jnp.exp(-d2), 0.0)
                return acc

            pull = jnp.broadcast_to(jnp.sum(pull_acc), (16,))
            push = jnp.broadcast_to(jnp.sum(push_acc), (16,))
            n = jnp.broadcast_to(jnp.sum(nval_acc), (16,))
            push_o = jnp.where(n > 0.0, push / ((n - 1.0) * n + _EPS), 0.0)
            pull_o = jnp.where(n > 0.0, pull / (n + _EPS), 0.0)
            row_v[...] = jnp.where(lanes == 0, push_o,
                                   jnp.where(lanes == 1, pull_o, 0.0))
            pltpu.sync_copy(row_v, out_ref.at[img])

    return pl.kernel(
        body,
        out_type=jax.ShapeDtypeStruct((num_images, 16), jnp.float32),
        mesh=mesh,
        compiler_params=pltpu.CompilerParams(needs_layout_passes=False),
        scratch_types=[
            pltpu.VMEM((_KP_ROW,), jnp.int32),
            pltpu.VMEM((512,), jnp.int32),
            pltpu.VMEM((_GAT,), jnp.float32),
            pltpu.VMEM((32,), jnp.float32),
            pltpu.VMEM((32,), jnp.float32),
            pltpu.VMEM((16,), jnp.float32),
            pltpu.SemaphoreType.DMA,
        ],
    )


def kernel(tags, keypoints):
    num_images, tags_per_image, _ = tags.shape
    tags2 = tags.reshape(num_images * tags_per_image)
    kp2 = keypoints.reshape(num_images, _KP_WORDS)
    kp_pad = jnp.zeros((num_images, _KP_ROW), jnp.int32)
    kp_pad = kp_pad.at[:, :_KP_WORDS].set(kp2)
    out = _build(num_images, tags_per_image)(tags2, kp_pad)
    return out[:, :2]


# speedup vs baseline: 1.0605x; 1.0605x over previous
import functools

import jax
import jax.numpy as jnp
from jax import lax
from jax.experimental import pallas as pl
from jax.experimental.pallas import tpu as pltpu
from jax.experimental.pallas import tpu_sc as plsc

_PEOPLE = 30
_JOINTS = 17
_PPAD = 64
_KP_ROW = _PPAD * _JOINTS * 2
_KP = _PEOPLE * _JOINTS
_EPS = 1e-6


@functools.lru_cache(maxsize=None)
def _build(num_images, tags_per_image):
    mesh = plsc.VectorSubcoreMesh(core_axis_name="c", subcore_axis_name="s")
    per_core = num_images // 2

    def body(tags_ref, kp_ref, out_ref, kp_v, idx_v, gat_v,
             mean_v, val_v, row_v, sem):
        cid = lax.axis_index("c")
        sid = lax.axis_index("s")
        img = cid * per_core + sid
        lanes = lax.iota(jnp.int32, 16)

        @pl.when(sid < per_core)
        def _():
            pltpu.sync_copy(kp_ref.at[img], kp_v)

            off = jnp.full((16,), img * tags_per_image, jnp.int32)

            @pl.loop(0, 32)
            def _(c):
                flat = lanes + c * 16
                a_idx = jnp.minimum(flat * 2, 2 * _KP - 2)
                kv = plsc.load_gather(kp_v, [a_idx])
                idx_v[pl.ds(c * 16, 16)] = kv + off

            copies = []
            for b in range(4):
                copies.append(pltpu.async_copy(
                    tags_ref.at[idx_v.at[pl.ds(b * 128, 128)]],
                    gat_v.at[pl.ds(b * 128, 128)], sem))
            for cp in copies:
                cp.wait()

            pull_acc = jnp.zeros((16,), jnp.float32)
            nval_acc = jnp.zeros((16,), jnp.float32)
            fz = jnp.zeros((16,), jnp.float32)
            for pv in range(2):
                p0 = lanes + pv * 16
                pmask = p0 < _PEOPLE
                base = p0 * _JOINTS

                @pl.loop(0, _JOINTS, init_carry=(fz, fz, fz))
                def sums(j, carry):
                    s1, s2, cnt = carry
                    g = plsc.load_gather(gat_v, [base + j])
                    vi = plsc.load_gather(kp_v, [(base + j) * 2 + 1])
                    vb = (vi > 0) & pmask
                    s1 = s1 + jnp.where(vb, g, 0.0)
                    s2 = s2 + jnp.where(vb, g * g, 0.0)
                    cnt = cnt + jnp.where(vb, 1.0, 0.0)
                    return s1, s2, cnt

                s1, s2, cnt = sums
                safe = jnp.maximum(cnt, 1.0)
                mean = s1 / safe
                valid = cnt > 0.0
                pull_acc = pull_acc + jnp.where(
                    valid, (s2 - mean * s1) / safe, 0.0)
                nval_acc = nval_acc + jnp.where(valid, 1.0, 0.0)
                mean_v[pl.ds(pv * 16, 16)] = mean
                val_v[pl.ds(pv * 16, 16)] = jnp.where(valid, 1.0, 0.0)

            @pl.loop(1, _PEOPLE, init_carry=fz)
            def push_acc(q, acc):
                qs = jnp.broadcast_to(q, (16,))
                mq = plsc.load_gather(mean_v, [qs])
                vq = plsc.load_gather(val_v, [qs])
                for pv in range(2):
                    p0 = lanes + pv * 16
                    mp = mean_v[pl.ds(pv * 16, 16)]
                    vp = val_v[pl.ds(pv * 16, 16)]
                    d2 = (mp - mq) * (mp - mq)
                    sel = ((p0 < q) & (d2 != 0.0)
                           & (vp > 0.0) & (vq > 0.0))
                    acc = acc + jnp.where(sel, jnp.exp(-d2), 0.0)
                return acc

            pull = jnp.broadcast_to(jnp.sum(pull_acc), (16,))
            push = jnp.broadcast_to(jnp.sum(push_acc), (16,))
            n = jnp.broadcast_to(jnp.sum(nval_acc), (16,))
            push_o = jnp.where(n > 0.0, push / ((n - 1.0) * n + _EPS), 0.0)
            pull_o = jnp.where(n > 0.0, pull / (n + _EPS), 0.0)
            row_v[...] = jnp.where(lanes == 0, push_o,
                                   jnp.where(lanes == 1, pull_o, 0.0))
            pltpu.sync_copy(row_v, out_ref.at[img])

    return pl.kernel(
        body,
        out_type=jax.ShapeDtypeStruct((num_images, 16), jnp.float32),
        mesh=mesh,
        compiler_params=pltpu.CompilerParams(needs_layout_passes=False),
        scratch_types=[
            pltpu.VMEM((_KP_ROW,), jnp.int32),
            pltpu.VMEM((512,), jnp.int32),
            pltpu.VMEM((544,), jnp.float32),
            pltpu.VMEM((32,), jnp.float32),
            pltpu.VMEM((32,), jnp.float32),
            pltpu.VMEM((16,), jnp.float32),
            pltpu.SemaphoreType.DMA,
        ],
    )


def kernel(tags, keypoints):
    num_images, tags_per_image, _ = tags.shape
    tags_flat = tags.reshape(num_images * tags_per_image)
    kp_pad = jnp.pad(
        keypoints, ((0, 0), (0, _PPAD - _PEOPLE), (0, 0), (0, 0)))
    kp2 = kp_pad.reshape(num_images, _KP_ROW)
    out = _build(num_images, tags_per_image)(tags_flat, kp2)
    return out[:, :2]


# speedup vs baseline: 1.1331x; 1.0685x over previous
import functools

import jax
import jax.numpy as jnp
from jax import lax
from jax.experimental import pallas as pl
from jax.experimental.pallas import tpu as pltpu
from jax.experimental.pallas import tpu_sc as plsc

_PEOPLE = 30
_JOINTS = 17
_PPAD = 64
_KP_ROW = _PPAD * _JOINTS * 2
_KP = _PEOPLE * _JOINTS
_EPS = 1e-6


@functools.lru_cache(maxsize=None)
def _build(num_images, tags_per_image):
    mesh = plsc.VectorSubcoreMesh(
        core_axis_name="c", subcore_axis_name="s", num_cores=1)
    per_core = num_images

    def body(tags_ref, kp_ref, out_ref, kp_v, idx_v, gat_v,
             mean_v, val_v, row_v, sem):
        cid = lax.axis_index("c")
        sid = lax.axis_index("s")
        img = cid * per_core + sid
        lanes = lax.iota(jnp.int32, 16)

        @pl.when(sid < per_core)
        def _():
            pltpu.sync_copy(kp_ref.at[img], kp_v)

            off = jnp.full((16,), img * tags_per_image, jnp.int32)

            @pl.loop(0, 32)
            def _(c):
                flat = lanes + c * 16
                a_idx = jnp.minimum(flat * 2, 2 * _KP - 2)
                kv = plsc.load_gather(kp_v, [a_idx])
                idx_v[pl.ds(c * 16, 16)] = kv + off

            copies = []
            for b in range(4):
                copies.append(pltpu.async_copy(
                    tags_ref.at[idx_v.at[pl.ds(b * 128, 128)]],
                    gat_v.at[pl.ds(b * 128, 128)], sem))
            for cp in copies:
                cp.wait()

            pull_acc = jnp.zeros((16,), jnp.float32)
            nval_acc = jnp.zeros((16,), jnp.float32)
            fz = jnp.zeros((16,), jnp.float32)
            for pv in range(2):
                p0 = lanes + pv * 16
                pmask = p0 < _PEOPLE
                base = p0 * _JOINTS

                @pl.loop(0, _JOINTS, init_carry=(fz, fz, fz))
                def sums(j, carry):
                    s1, s2, cnt = carry
                    g = plsc.load_gather(gat_v, [base + j])
                    vi = plsc.load_gather(kp_v, [(base + j) * 2 + 1])
                    vb = (vi > 0) & pmask
                    s1 = s1 + jnp.where(vb, g, 0.0)
                    s2 = s2 + jnp.where(vb, g * g, 0.0)
                    cnt = cnt + jnp.where(vb, 1.0, 0.0)
                    return s1, s2, cnt

                s1, s2, cnt = sums
                safe = jnp.maximum(cnt, 1.0)
                mean = s1 / safe
                valid = cnt > 0.0
                pull_acc = pull_acc + jnp.where(
                    valid, (s2 - mean * s1) / safe, 0.0)
                nval_acc = nval_acc + jnp.where(valid, 1.0, 0.0)
                mean_v[pl.ds(pv * 16, 16)] = mean
                val_v[pl.ds(pv * 16, 16)] = jnp.where(valid, 1.0, 0.0)

            @pl.loop(1, _PEOPLE, init_carry=fz)
            def push_acc(q, acc):
                qs = jnp.broadcast_to(q, (16,))
                mq = plsc.load_gather(mean_v, [qs])
                vq = plsc.load_gather(val_v, [qs])
                for pv in range(2):
                    p0 = lanes + pv * 16
                    mp = mean_v[pl.ds(pv * 16, 16)]
                    vp = val_v[pl.ds(pv * 16, 16)]
                    d2 = (mp - mq) * (mp - mq)
                    sel = ((p0 < q) & (d2 != 0.0)
                           & (vp > 0.0) & (vq > 0.0))
                    acc = acc + jnp.where(sel, jnp.exp(-d2), 0.0)
                return acc

            pull = jnp.broadcast_to(jnp.sum(pull_acc), (16,))
            push = jnp.broadcast_to(jnp.sum(push_acc), (16,))
            n = jnp.broadcast_to(jnp.sum(nval_acc), (16,))
            push_o = jnp.where(n > 0.0, push / ((n - 1.0) * n + _EPS), 0.0)
            pull_o = jnp.where(n > 0.0, pull / (n + _EPS), 0.0)
            row_v[...] = jnp.where(lanes == 0, push_o,
                                   jnp.where(lanes == 1, pull_o, 0.0))
            pltpu.sync_copy(row_v, out_ref.at[img])

    return pl.kernel(
        body,
        out_type=jax.ShapeDtypeStruct((num_images, 16), jnp.float32),
        mesh=mesh,
        compiler_params=pltpu.CompilerParams(needs_layout_passes=False),
        scratch_types=[
            pltpu.VMEM((_KP_ROW,), jnp.int32),
            pltpu.VMEM((512,), jnp.int32),
            pltpu.VMEM((544,), jnp.float32),
            pltpu.VMEM((32,), jnp.float32),
            pltpu.VMEM((32,), jnp.float32),
            pltpu.VMEM((16,), jnp.float32),
            pltpu.SemaphoreType.DMA,
        ],
    )


def kernel(tags, keypoints):
    num_images, tags_per_image, _ = tags.shape
    tags_flat = tags.reshape(num_images * tags_per_image)
    kp_pad = jnp.pad(
        keypoints, ((0, 0), (0, _PPAD - _PEOPLE), (0, 0), (0, 0)))
    kp2 = kp_pad.reshape(num_images, _KP_ROW)
    out = _build(num_images, tags_per_image)(tags_flat, kp2)
    return out[:, :2]
